# baseline (speedup 1.0000x reference)
import functools

import jax
import jax.numpy as jnp
from jax import lax
from jax.experimental import pallas as pl
from jax.experimental.pallas import tpu as pltpu
from jax.experimental.pallas import tpu_sc as plsc

NUM_CLASSES = 1_000_000
HIDDEN = 64
BATCH = 16384

_NC = 2
_NS = 16
_NW = _NC * _NS

_B_PER_W = BATCH // _NW


@functools.partial(
    pl.kernel,
    out_type=jax.ShapeDtypeStruct((BATCH, HIDDEN), jnp.float32),
    mesh=plsc.VectorSubcoreMesh(core_axis_name="c", subcore_axis_name="s"),
    scratch_types=[
        pltpu.VMEM((_B_PER_W,), jnp.int32),
        pltpu.VMEM((_B_PER_W, HIDDEN), jnp.float32),
        pltpu.SemaphoreType.DMA,
    ],
)
def _gather_kernel(table_hbm, idx_hbm, out_hbm, labels_v, rowbuf, sem):
    wid = lax.axis_index("s") * _NC + lax.axis_index("c")
    base = wid * _B_PER_W
    pltpu.sync_copy(idx_hbm.at[pl.ds(base, _B_PER_W)], labels_v)

    copies = []
    for h in range(_B_PER_W // 16):
        lvec = labels_v[pl.ds(h * 16, 16)]
        for g in range(16):
            i = h * 16 + g
            lab = lvec[g]
            copies.append(
                pltpu.async_copy(table_hbm.at[lab], out_hbm.at[base + i],
                                 sem))
    for c in copies:
        c.wait()


def kernel(labels, embedding_table):
    return _gather_kernel(embedding_table, labels.astype(jnp.int32))
